# baseline (speedup 1.0000x reference)
import functools

import jax
import jax.numpy as jnp
from jax.experimental import pallas as pl
from jax.experimental.pallas import tpu as pltpu

_K = 16


def _fused_body(sim_ref, mask_ref, feat_ref, out_ref,
                vvals0, vidx0, vvals1, vidx1, svals, sidx, semv, semi,
                *, tm, nblocks, n):
    i = pl.program_id(0)
    par = i & 1

    @pl.when(jnp.logical_and(i >= 2, par == 0))
    def _():
        pltpu.make_async_copy(vvals0, svals.at[0], semv.at[0]).wait()
        pltpu.make_async_copy(vidx0, sidx.at[0], semi.at[0]).wait()

    @pl.when(jnp.logical_and(i >= 2, par == 1))
    def _():
        pltpu.make_async_copy(vvals1, svals.at[1], semv.at[1]).wait()
        pltpu.make_async_copy(vidx1, sidx.at[1], semi.at[1]).wait()

    nmask = n - 1

    def _gather_rows(r0, r1):
        for r in range(r0, r1):
            iv = sidx[par, r, 0] & nmask
            acc = svals[par, r, 0] * feat_ref[iv]
            for j in range(1, _K):
                ivj = sidx[par, r, j] & nmask
                acc = acc + svals[par, r, j] * feat_ref[ivj]
            out_ref[r] = acc

    seg = tm // _K

    nchunks = max(sim_ref.shape[1] // 128, 1)
    x = sim_ref[...]
    ui = pltpu.bitcast(x, jnp.uint32)
    col = jax.lax.broadcasted_iota(jnp.uint32, x.shape, 1)
    chunk_rev = jnp.uint32(nchunks - 1) - (col >> 7)
    enc = (ui & jnp.uint32(0xFFFFFFE0)) | chunk_rev
    sim_ref[...] = pltpu.bitcast(enc, jnp.float32)
    vcols, icols = [], []
    for p in range(_K):
        _gather_rows(p * seg, (p + 1) * seg)
        y = sim_ref[...]
        tree = y[:, :128]
        for c in range(1, nchunks):
            tree = jnp.maximum(tree, y[:, c * 128:(c + 1) * 128])
        m = jnp.max(tree, axis=1, keepdims=True)
        lane = jnp.argmax(tree, axis=1).astype(jnp.int32)[:, None]
        sim_ref[...] = jnp.where(y == m, -1.0, y)
        mui = pltpu.bitcast(m, jnp.uint32)
        chunk = jnp.int32(nchunks - 1) - (mui & jnp.uint32(31)).astype(jnp.int32)
        icols.append(chunk * 128 + lane)
        vcols.append(pltpu.bitcast(mui & jnp.uint32(0xFFFFFFE0), jnp.float32))
    vals = jnp.concatenate(vcols, axis=1)
    idx = jnp.concatenate(icols, axis=1)
    denom = jnp.sum(vals, axis=1, keepdims=True)
    mask = mask_ref[...] > 0.0
    scaled = jnp.where(mask, 0.0, vals / denom)
    kcol = jax.lax.broadcasted_iota(jnp.int32, vals.shape, 1)
    scaled = jnp.where(mask & (kcol == 0), 1.0, scaled)
    base = i * tm
    rows = base + jax.lax.broadcasted_iota(jnp.int32, idx.shape, 0)
    idxm = jnp.where(mask, rows, idx)

    @pl.when(jnp.logical_and(i < nblocks, par == 0))
    def _():
        vvals0[...] = scaled
        vidx0[...] = idxm
        pltpu.make_async_copy(vvals0, svals.at[0], semv.at[0]).start()
        pltpu.make_async_copy(vidx0, sidx.at[0], semi.at[0]).start()

    @pl.when(jnp.logical_and(i < nblocks, par == 1))
    def _():
        vvals1[...] = scaled
        vidx1[...] = idxm
        pltpu.make_async_copy(vvals1, svals.at[1], semv.at[1]).start()
        pltpu.make_async_copy(vidx1, sidx.at[1], semi.at[1]).start()


def _smooth_fused(sim, mask_f, feat3, *, tm=256):
    n, w = sim.shape
    tm = min(tm, n)
    nblocks = n // tm
    grid = (nblocks + 2,)
    nb1 = nblocks - 1
    return pl.pallas_call(
        functools.partial(_fused_body, tm=tm, nblocks=nblocks, n=n),
        out_shape=jax.ShapeDtypeStruct((n, 1, feat3.shape[2]), jnp.float32),
        grid=grid,
        in_specs=[
            pl.BlockSpec((tm, w), lambda i: (jnp.minimum(i, nb1), 0)),
            pl.BlockSpec((tm, 1), lambda i: (jnp.minimum(i, nb1), 0)),
            pl.BlockSpec((n, 1, feat3.shape[2]), lambda i: (0, 0, 0)),
        ],
        out_specs=pl.BlockSpec((tm, 1, feat3.shape[2]),
                               lambda i: (jnp.maximum(i - 2, 0), 0, 0)),
        scratch_shapes=[
            pltpu.VMEM((tm, _K), jnp.float32),
            pltpu.VMEM((tm, _K), jnp.int32),
            pltpu.VMEM((tm, _K), jnp.float32),
            pltpu.VMEM((tm, _K), jnp.int32),
            pltpu.SMEM((2, tm, _K), jnp.float32),
            pltpu.SMEM((2, tm, _K), jnp.int32),
            pltpu.SemaphoreType.DMA((2,)),
            pltpu.SemaphoreType.DMA((2,)),
        ],
        compiler_params=pltpu.CompilerParams(
            dimension_semantics=("arbitrary",),
            vmem_limit_bytes=48 * 1024 * 1024),
    )(sim, mask_f, feat3)


def _decode_body(u_ref, v_ref, out_ref):
    x = jax.lax.dot_general(u_ref[...], v_ref[...],
                            dimension_numbers=(((1,), (1,)), ((), ())),
                            preferred_element_type=jnp.float32)
    out_ref[...] = jax.nn.sigmoid(x)


def _decode(u, v, *, tm=256, tn=512):
    su, d = u.shape
    sv, _ = v.shape
    tm = min(tm, su)
    tn = min(tn, sv)
    grid = (su // tm, sv // tn)
    return pl.pallas_call(
        _decode_body,
        out_shape=jax.ShapeDtypeStruct((su, sv), jnp.float32),
        grid=grid,
        in_specs=[
            pl.BlockSpec((tm, d), lambda i, j: (i, 0)),
            pl.BlockSpec((tn, d), lambda i, j: (j, 0)),
        ],
        out_specs=pl.BlockSpec((tm, tn), lambda i, j: (i, j)),
        compiler_params=pltpu.CompilerParams(
            dimension_semantics=("parallel", "parallel"),
            vmem_limit_bytes=48 * 1024 * 1024),
    )(u, v)


def kernel(u, v, sim, mask_bool):
    size_u, d = u.shape
    feature = jnp.concatenate([u, v], axis=0).astype(jnp.float32)
    n = feature.shape[0]

    mask_f = mask_bool.reshape(n, 1).astype(jnp.float32)
    out3 = _smooth_fused(sim, mask_f, feature.reshape(n, 1, d))
    smoothed = out3.reshape(n, d)
    u_new = smoothed[:size_u]
    v_new = smoothed[size_u:]

    outputs = _decode(u_new.astype(jnp.bfloat16), v_new.astype(jnp.bfloat16))
    return outputs, u_new, v_new


# speedup vs baseline: 1.1198x; 1.1198x over previous
import functools

import jax
import jax.numpy as jnp
from jax.experimental import pallas as pl
from jax.experimental.pallas import tpu as pltpu

_K = 16


def _topk_body(sim_ref, mask_ref, idx_ref, val_ref, *, tm):
    nchunks = max(sim_ref.shape[1] // 128, 1)
    x = sim_ref[...]
    ui = pltpu.bitcast(x, jnp.uint32)
    col = jax.lax.broadcasted_iota(jnp.uint32, x.shape, 1)
    chunk_rev = jnp.uint32(nchunks - 1) - (col >> 7)
    enc = (ui & jnp.uint32(0xFFFFFFE0)) | chunk_rev
    y0 = pltpu.bitcast(enc, jnp.float32)
    vcols, icols = [], []
    for p in range(_K):
        y = y0 if p == 0 else sim_ref[...]
        tree = y[:, :128]
        for c in range(1, nchunks):
            tree = jnp.maximum(tree, y[:, c * 128:(c + 1) * 128])
        m = jnp.max(tree, axis=1, keepdims=True)
        lane = jnp.argmax(tree, axis=1).astype(jnp.int32)[:, None]
        sim_ref[...] = jnp.where(y == m, -1.0, y)
        mui = pltpu.bitcast(m, jnp.uint32)
        chunk = jnp.int32(nchunks - 1) - (mui & jnp.uint32(31)).astype(jnp.int32)
        icols.append(chunk * 128 + lane)
        vcols.append(pltpu.bitcast(mui & jnp.uint32(0xFFFFFFE0), jnp.float32))
    vals = jnp.concatenate(vcols, axis=1)
    idx = jnp.concatenate(icols, axis=1)
    denom = jnp.sum(vals, axis=1, keepdims=True)
    mask = mask_ref[...] > 0.0
    scaled = jnp.where(mask, 0.0, vals / denom)
    kcol = jax.lax.broadcasted_iota(jnp.int32, vals.shape, 1)
    scaled = jnp.where(mask & (kcol == 0), 1.0, scaled)
    base = pl.program_id(0) * tm
    rows = base + jax.lax.broadcasted_iota(jnp.int32, idx.shape, 0)
    val_ref[...] = scaled
    idx_ref[...] = jnp.where(mask, rows, idx)


def _topk(sim, mask_f, *, tm=512):
    n, w = sim.shape
    tm = min(tm, n)
    grid = (n // tm,)
    return pl.pallas_call(
        functools.partial(_topk_body, tm=tm),
        out_shape=(jax.ShapeDtypeStruct((n, _K), jnp.int32),
                   jax.ShapeDtypeStruct((n, _K), jnp.float32)),
        grid=grid,
        in_specs=[
            pl.BlockSpec((tm, w), lambda i: (i, 0)),
            pl.BlockSpec((tm, 1), lambda i: (i, 0)),
        ],
        out_specs=(pl.BlockSpec((tm, _K), lambda i: (i, 0)),
                   pl.BlockSpec((tm, _K), lambda i: (i, 0))),
        compiler_params=pltpu.CompilerParams(
            dimension_semantics=("parallel",),
            vmem_limit_bytes=48 * 1024 * 1024),
    )(sim, mask_f)


def _smooth_body(idx_ref, val_ref, feat_ref, out_ref, *, tm, unroll=8):
    def chunk(it, carry):
        r0 = it * unroll
        accs = []
        for uu in range(unroll):
            r = r0 + uu
            acc = val_ref[r, 0] * feat_ref[idx_ref[r, 0]]
            for j in range(1, _K):
                acc = acc + val_ref[r, j] * feat_ref[idx_ref[r, j]]
            accs.append(acc)
        for uu in range(unroll):
            out_ref[r0 + uu] = accs[uu]
        return carry

    jax.lax.fori_loop(0, tm // unroll, chunk, 0)


def _smooth(idx, values, feat3, *, tm=256):
    n, _, d = feat3.shape
    tm = min(tm, n)
    grid = (n // tm,)
    return pl.pallas_call(
        functools.partial(_smooth_body, tm=tm),
        out_shape=jax.ShapeDtypeStruct((n, 1, d), jnp.float32),
        grid=grid,
        in_specs=[
            pl.BlockSpec((tm, _K), lambda i: (i, 0), memory_space=pltpu.SMEM),
            pl.BlockSpec((tm, _K), lambda i: (i, 0), memory_space=pltpu.SMEM),
            pl.BlockSpec((n, 1, d), lambda i: (0, 0, 0)),
        ],
        out_specs=pl.BlockSpec((tm, 1, d), lambda i: (i, 0, 0)),
        compiler_params=pltpu.CompilerParams(
            dimension_semantics=("parallel",),
            vmem_limit_bytes=48 * 1024 * 1024),
    )(idx, values, feat3)


def _decode_body(u_ref, v_ref, out_ref):
    x = jax.lax.dot_general(u_ref[...], v_ref[...],
                            dimension_numbers=(((1,), (1,)), ((), ())),
                            preferred_element_type=jnp.float32)
    out_ref[...] = 0.5 + 0.5 * jnp.tanh(0.5 * x)


def _decode(u, v, *, tm=256, tn=512):
    su, d = u.shape
    sv, _ = v.shape
    tm = min(tm, su)
    tn = min(tn, sv)
    grid = (su // tm, sv // tn)
    return pl.pallas_call(
        _decode_body,
        out_shape=jax.ShapeDtypeStruct((su, sv), jnp.float32),
        grid=grid,
        in_specs=[
            pl.BlockSpec((tm, d), lambda i, j: (i, 0)),
            pl.BlockSpec((tn, d), lambda i, j: (j, 0)),
        ],
        out_specs=pl.BlockSpec((tm, tn), lambda i, j: (i, j)),
        compiler_params=pltpu.CompilerParams(
            dimension_semantics=("parallel", "parallel"),
            vmem_limit_bytes=48 * 1024 * 1024),
    )(u, v)


def kernel(u, v, sim, mask_bool):
    size_u, d = u.shape
    feature = jnp.concatenate([u, v], axis=0).astype(jnp.float32)
    n = feature.shape[0]

    mask_f = mask_bool.reshape(n, 1).astype(jnp.float32)
    idx, scaled = _topk(sim, mask_f)

    out3 = _smooth(idx, scaled, feature.reshape(n, 1, d))
    smoothed = out3.reshape(n, d)
    u_new = smoothed[:size_u]
    v_new = smoothed[size_u:]

    outputs = _decode(u_new.astype(jnp.bfloat16), v_new.astype(jnp.bfloat16))
    return outputs, u_new, v_new
